# baseline (speedup 1.0000x reference)
import functools

import jax
import jax.numpy as jnp
from jax import lax
from jax.experimental import pallas as pl
from jax.experimental.pallas import tpu as pltpu
from jax.experimental.pallas import tpu_sc as plsc

N1 = 50000
N1P = 51200
DUMP = 50000
U = 128
NCOL = 4
CW = U // NCOL
ROWS_PER_TILE = N1P // 16

GDEFS = {"g2": (2, 50000, 51200), "g3": (3, 80000, 81920), "g4": (4, 100000, 100352)}
KG = 2
KS = 1

_MESH = dict(core_axis_name="c", subcore_axis_name="s")


def _dot(a, b):
    return jnp.dot(a, b, preferred_element_type=jnp.float32,
                   precision=lax.Precision.HIGHEST)


def _sc_gather(table, idx2d):
    NB = idx2d.shape[0]
    nper = NB // 32
    nmain = nper // (2 * KG)
    ntail0 = nmain * 2 * KG
    W = KG * 128

    @functools.partial(
        pl.kernel,
        out_type=jax.ShapeDtypeStruct((NB * 128, U), jnp.float32),
        mesh=plsc.VectorSubcoreMesh(**_MESH),
        scratch_types=[
            pltpu.VMEM((nper, 128), jnp.int32),
            pltpu.VMEM((W, U), jnp.float32),
            pltpu.VMEM((W, U), jnp.float32),
            pltpu.SemaphoreType.DMA,
            pltpu.SemaphoreType.DMA,
            pltpu.SemaphoreType.DMA,
            pltpu.SemaphoreType.DMA,
        ],
        compiler_params=pltpu.CompilerParams(use_tc_tiling_on_sc=False),
    )
    def k(table_hbm, idx_hbm, out_hbm, idx_v, rows0, rows1, g0, g1, o0, o1):
        wid = lax.axis_index("s") * 2 + lax.axis_index("c")
        b0 = wid * nper
        pltpu.sync_copy(idx_hbm.at[pl.ds(b0, nper)], idx_v)
        rows = (rows0, rows1)
        gsem = (g0, g1)
        osem = (o0, o1)

        @pl.loop(0, nmain)
        def _(jo):
            for d in range(2):
                base = (jo * 2 + d) * KG

                @pl.when(jo > 0)
                def _():
                    pltpu.make_async_copy(
                        rows[d], out_hbm.at[pl.ds(b0 * 128, W)], osem[d]).wait()

                for kk in range(KG):
                    pltpu.async_copy(table_hbm.at[idx_v.at[base + kk]],
                                     rows[d].at[pl.ds(kk * 128, 128)], gsem[d])
            for d in range(2):
                base = (jo * 2 + d) * KG
                for kk in range(KG):
                    pltpu.make_async_copy(
                        table_hbm.at[idx_v.at[kk]],
                        rows[d].at[pl.ds(kk * 128, 128)], gsem[d]).wait()
                pltpu.async_copy(rows[d],
                                 out_hbm.at[pl.ds((b0 + base) * 128, W)], osem[d])

        if nmain > 0:
            for d in range(2):
                pltpu.make_async_copy(
                    rows[d], out_hbm.at[pl.ds(b0 * 128, W)], osem[d]).wait()

        if ntail0 < nper:
            @pl.loop(ntail0, nper)
            def _(b):
                pltpu.async_copy(table_hbm.at[idx_v.at[b]],
                                 rows0.at[pl.ds(0, 128)], g0).wait()
                pltpu.sync_copy(rows0.at[pl.ds(0, 128)],
                                out_hbm.at[pl.ds((b0 + b) * 128, 128)])

    return k(table, idx2d)


def _sc_scatter_add(rows3, idx2d, zeros_tile):
    nbs = [r.shape[0] // 128 for r in rows3]
    assert sum(nbs) == idx2d.shape[0]
    W = KS * 128

    @functools.partial(
        pl.kernel,
        out_type=jax.ShapeDtypeStruct((N1P, U), jnp.float32),
        mesh=plsc.VectorSubcoreMesh(**_MESH),
        scratch_types=[
            pltpu.VMEM((KS, 128), jnp.int32),
            pltpu.VMEM((KS, 128), jnp.int32),
            pltpu.VMEM((W, CW), jnp.float32),
            pltpu.VMEM((W, CW), jnp.float32),
            pltpu.VMEM_SHARED((N1P, CW), jnp.float32),
            pltpu.SemaphoreType.DMA,
            pltpu.SemaphoreType.DMA,
            pltpu.SemaphoreType.DMA,
            pltpu.SemaphoreType.DMA,
        ],
        compiler_params=pltpu.CompilerParams(use_tc_tiling_on_sc=False),
    )
    def k(r2_hbm, r3_hbm, r4_hbm, idx_hbm, zeros_hbm, acc_hbm,
          idx0, idx1, rows0, rows1, acc_sh, c0, c1, a0, a1):
        c = lax.axis_index("c")
        s = lax.axis_index("s")
        idxs_v = (idx0, idx1)
        rows_v = (rows0, rows1)
        csem = (c0, c1)
        asem = (a0, a1)
        for p in range(2):
            col0 = (c * 2 + p) * CW
            pltpu.sync_copy(zeros_hbm, acc_sh.at[pl.ds(s * ROWS_PER_TILE, ROWS_PER_TILE)])
            plsc.subcore_barrier()

            nboff = 0
            for rows_hbm, nb in zip((r2_hbm, r3_hbm, r4_hbm), nbs):
                nper = nb // 16
                nmain = nper // (2 * KS)
                b0 = nboff + s * nper
                r0 = s * nper

                @pl.loop(0, nmain)
                def _(j):
                    ld = []
                    for d in range(2):
                        bb = (j * 2 + d) * KS
                        ld.append((
                            pltpu.async_copy(idx_hbm.at[pl.ds(b0 + bb, KS)],
                                             idxs_v[d], csem[d]),
                            pltpu.async_copy(
                                rows_hbm.at[pl.ds((r0 + bb) * 128, W),
                                            pl.ds(col0, CW)],
                                rows_v[d], csem[d]),
                        ))
                    adds = []
                    for d in range(2):
                        for cp in ld[d]:
                            cp.wait()
                        adds.extend(
                            pltpu.async_copy(rows_v[d].at[pl.ds(kk * 128, 128)],
                                             acc_sh.at[idxs_v[d].at[kk]],
                                             asem[d], add=True)
                            for kk in range(KS))
                    for cp in adds:
                        cp.wait()

                if nmain * 2 * KS < nper:
                    @pl.loop(nmain * 2 * KS, nper)
                    def _(b):
                        pltpu.sync_copy(idx_hbm.at[pl.ds(b0 + b, 1)],
                                        idx0.at[pl.ds(0, 1)])
                        pltpu.sync_copy(
                            rows_hbm.at[pl.ds((r0 + b) * 128, 128), pl.ds(col0, CW)],
                            rows0.at[pl.ds(0, 128)])
                        pltpu.async_copy(rows0.at[pl.ds(0, 128)],
                                         acc_sh.at[idx0.at[0]], a0, add=True).wait()
                nboff += nb

            plsc.subcore_barrier()
            pltpu.sync_copy(
                acc_sh.at[pl.ds(s * ROWS_PER_TILE, ROWS_PER_TILE)],
                acc_hbm.at[pl.ds(s * ROWS_PER_TILE, ROWS_PER_TILE), pl.ds(col0, CW)],
            )
            plsc.subcore_barrier()

    return k(rows3[0], rows3[1], rows3[2], idx2d, zeros_tile)


_R = 512


def _tc_fin(h0p, w, b):
    def body(x_ref, w_ref, b_ref, o_ref):
        o_ref[...] = jnp.tanh(_dot(x_ref[...], w_ref[...]) + b_ref[...])

    return pl.pallas_call(
        body,
        grid=(N1P // _R,),
        in_specs=[
            pl.BlockSpec((_R, U), lambda i: (i, 0)),
            pl.BlockSpec((U, U), lambda i: (0, 0)),
            pl.BlockSpec((1, U), lambda i: (0, 0)),
        ],
        out_specs=pl.BlockSpec((_R, U), lambda i: (i, 0)),
        out_shape=jax.ShapeDtypeStruct((N1P, U), jnp.float32),
    )(h0p, w, b)


def _tc_gru(m, wih, whh, bih, bhh, wd1g, T, ngp, want_hg):
    def body(m_ref, wih_ref, whh_ref, bih_ref, bhh_ref, wd1_ref, y_ref,
             *maybe_hg):
        wih_v = wih_ref[...]
        whh_v = whh_ref[...]
        bih_v = bih_ref[...]
        bhh_v = bhh_ref[...]
        wd1_v = wd1_ref[...]
        h = None
        for t in range(T):
            gi = _dot(m_ref[t], wih_v) + bih_v
            gh = bhh_v if h is None else _dot(h, whh_v) + bhh_v
            r = jax.nn.sigmoid(gi[:, 0:U] + gh[..., 0:U])
            z = jax.nn.sigmoid(gi[:, U:2 * U] + gh[..., U:2 * U])
            n = jnp.tanh(gi[:, 2 * U:] + r * gh[..., 2 * U:])
            h = n - z * n if h is None else (1.0 - z) * n + z * h
            y_ref[t] = _dot(h, wd1_v)
        if want_hg:
            maybe_hg[0][...] = h

    out_shape = [jax.ShapeDtypeStruct((T, ngp, U), jnp.float32)]
    out_specs = [pl.BlockSpec((T, _R, U), lambda i: (0, i, 0))]
    if want_hg:
        out_shape.append(jax.ShapeDtypeStruct((ngp, U), jnp.float32))
        out_specs.append(pl.BlockSpec((_R, U), lambda i: (i, 0)))
    return pl.pallas_call(
        body,
        grid=(ngp // _R,),
        in_specs=[
            pl.BlockSpec((T, _R, U), lambda i: (0, i, 0)),
            pl.BlockSpec((U, 3 * U), lambda i: (0, 0)),
            pl.BlockSpec((U, 3 * U), lambda i: (0, 0)),
            pl.BlockSpec((1, 3 * U), lambda i: (0, 0)),
            pl.BlockSpec((1, 3 * U), lambda i: (0, 0)),
            pl.BlockSpec((U, U), lambda i: (0, 0)),
        ],
        out_specs=out_specs,
        out_shape=out_shape,
    )(m, wih, whh, bih, bhh, wd1g)


def _tc_combine(h, acc, w_h, bd1, wd2, bd2):
    def body(h_ref, a_ref, wh_ref, b1_ref, wd2_ref, b2_ref, o_ref):
        t = _dot(h_ref[...], wh_ref[...]) + a_ref[...] + b1_ref[...]
        o_ref[...] = jnp.tanh(_dot(jnp.tanh(t), wd2_ref[...]) + b2_ref[...])

    rspec = pl.BlockSpec((_R, U), lambda i: (i, 0))
    wspec = pl.BlockSpec((U, U), lambda i: (0, 0))
    bspec = pl.BlockSpec((1, U), lambda i: (0, 0))
    return pl.pallas_call(
        body,
        grid=(N1P // _R,),
        in_specs=[rspec, rspec, wspec, bspec, wspec, bspec],
        out_specs=rspec,
        out_shape=jax.ShapeDtypeStruct((N1P, U), jnp.float32),
    )(h, acc, w_h, bd1, wd2, bd2)


def _tc_readout(x3, t_idx, w1, b1, w2p, b2p):
    np_rows = x3.shape[1]

    def body(x_ref, w1_ref, b1_ref, w2_ref, b2_ref, o_ref):
        t = _dot(x_ref[0], w1_ref[...]) + b1_ref[...]
        o_ref[...] = _dot(t, w2_ref[...]) + b2_ref[...]

    return pl.pallas_call(
        body,
        grid=(np_rows // _R,),
        in_specs=[
            pl.BlockSpec((1, _R, U), lambda i: (t_idx, i, 0)),
            pl.BlockSpec((U, U), lambda i: (0, 0)),
            pl.BlockSpec((1, U), lambda i: (0, 0)),
            pl.BlockSpec((U, 8), lambda i: (0, 0)),
            pl.BlockSpec((1, 8), lambda i: (0, 0)),
        ],
        out_specs=pl.BlockSpec((_R, 8), lambda i: (i, 0)),
        out_shape=jax.ShapeDtypeStruct((np_rows, 8), jnp.float32),
    )(x3, w1, b1, w2p, b2p)


def _layer(h, L, p, idx_all, zeros_tile, want_hg):
    wd1 = p[L + "_Wd1"]
    m_all = _sc_gather(h, idx_all)
    hgs = {}
    ys = []
    roff = 0
    for gi, (name, (T, _, ngp)) in enumerate(GDEFS.items()):
        m = m_all[roff:roff + T * ngp].reshape(T, ngp, U)
        roff += T * ngp
        outs = _tc_gru(m, p[L + "_Wih"], p[L + "_Whh"],
                       p[L + "_bih"][None, :], p[L + "_bhh"][None, :],
                       wd1[U * (gi + 1):U * (gi + 2)], T, ngp, want_hg)
        ys.append(outs[0].reshape(T * ngp, U))
        if want_hg:
            hgs[name] = outs[1]
    acc = _sc_scatter_add(ys, idx_all, zeros_tile)
    hnew = _tc_combine(h, acc, wd1[0:U],
                       p[L + "_bd1"][None, :], p[L + "_Wd2"], p[L + "_bd2"][None, :])
    return hnew, hgs


def kernel(h0, params, g2_idx, g3_idx, g4_idx):
    p = params
    idxs = {"g2": g2_idx, "g3": g3_idx, "g4": g4_idx}

    idx2ds = []
    for name, (T, ng, ngp) in GDEFS.items():
        it = jnp.full((T, ngp), DUMP, jnp.int32)
        it = it.at[:, :ng].set(idxs[name].astype(jnp.int32).T)
        idx2ds.append(it.reshape(-1, 128))
    idx_all = jnp.concatenate(idx2ds, axis=0)

    h0p = jnp.pad(h0, ((0, N1P - N1), (0, U - h0.shape[1])))
    finw = jnp.pad(p["fin_W"], ((0, U - p["fin_W"].shape[0]), (0, 0)))
    zeros_tile = jnp.zeros((ROWS_PER_TILE, CW), jnp.float32)

    h = _tc_fin(h0p, finw, p["fin_b"][None, :])
    h, _ = _layer(h, "d0", p, idx_all, zeros_tile, False)
    h, hgs = _layer(h, "d2", p, idx_all, zeros_tile, True)

    outs = []
    ro_in = {
        "atom": (h, N1),
        "bond": (hgs["g2"], N1),
        "angle": (hgs["g3"], GDEFS["g3"][1]),
        "torsion": (hgs["g4"], GDEFS["g4"][1]),
    }
    for term, (x2, nreal) in ro_in.items():
        w2p = jnp.pad(p["fr_" + term + "_W2"], ((0, 0), (0, 6)))
        b2p = jnp.pad(p["fr_" + term + "_b2"], ((0, 6)))[None, :]
        o = _tc_readout(x2[None], 0, p["fr_" + term + "_W1"],
                        p["fr_" + term + "_b1"][None, :], w2p, b2p)
        outs.append(o[:nreal, :2])
    return jnp.concatenate(outs, axis=0)


# speedup vs baseline: 1.2831x; 1.2831x over previous
import functools

import jax
import jax.numpy as jnp
from jax import lax
from jax.experimental import pallas as pl
from jax.experimental.pallas import tpu as pltpu
from jax.experimental.pallas import tpu_sc as plsc

N1 = 50000
N1P = 51200
DUMP = 50000
U = 128
NCOL = 4
CW = U // NCOL
ROWS_PER_TILE = N1P // 16

GDEFS = {"g2": (2, 50000, 51200), "g3": (3, 80000, 81920), "g4": (4, 100000, 100352)}
KG = 2
KS = 1

_MESH = dict(core_axis_name="c", subcore_axis_name="s")


def _dot(a, b):
    return jnp.dot(a, b, preferred_element_type=jnp.float32,
                   precision=lax.Precision.HIGHEST)


def _sc_gather(table, idx2d):
    NB = idx2d.shape[0]
    nper = NB // 32
    nmain = nper // (2 * KG)
    ntail0 = nmain * 2 * KG
    W = KG * 128

    @functools.partial(
        pl.kernel,
        out_type=jax.ShapeDtypeStruct((NB * 128, U), jnp.float32),
        mesh=plsc.VectorSubcoreMesh(**_MESH),
        scratch_types=[
            pltpu.VMEM((nper, 128), jnp.int32),
            pltpu.VMEM((W, U), jnp.float32),
            pltpu.VMEM((W, U), jnp.float32),
            pltpu.SemaphoreType.DMA,
            pltpu.SemaphoreType.DMA,
            pltpu.SemaphoreType.DMA,
            pltpu.SemaphoreType.DMA,
        ],
        compiler_params=pltpu.CompilerParams(use_tc_tiling_on_sc=False),
    )
    def k(table_hbm, idx_hbm, out_hbm, idx_v, rows0, rows1, g0, g1, o0, o1):
        wid = lax.axis_index("s") * 2 + lax.axis_index("c")
        b0 = wid * nper
        pltpu.sync_copy(idx_hbm.at[pl.ds(b0, nper)], idx_v)
        rows = (rows0, rows1)
        gsem = (g0, g1)
        osem = (o0, o1)

        @pl.loop(0, nmain)
        def _(jo):
            for d in range(2):
                base = (jo * 2 + d) * KG

                @pl.when(jo > 0)
                def _():
                    pltpu.make_async_copy(
                        rows[d], out_hbm.at[pl.ds(b0 * 128, W)], osem[d]).wait()

                for kk in range(KG):
                    pltpu.async_copy(table_hbm.at[idx_v.at[base + kk]],
                                     rows[d].at[pl.ds(kk * 128, 128)], gsem[d])
            for d in range(2):
                base = (jo * 2 + d) * KG
                for kk in range(KG):
                    pltpu.make_async_copy(
                        table_hbm.at[idx_v.at[kk]],
                        rows[d].at[pl.ds(kk * 128, 128)], gsem[d]).wait()
                pltpu.async_copy(rows[d],
                                 out_hbm.at[pl.ds((b0 + base) * 128, W)], osem[d])

        if nmain > 0:
            for d in range(2):
                pltpu.make_async_copy(
                    rows[d], out_hbm.at[pl.ds(b0 * 128, W)], osem[d]).wait()

        if ntail0 < nper:
            @pl.loop(ntail0, nper)
            def _(b):
                pltpu.async_copy(table_hbm.at[idx_v.at[b]],
                                 rows0.at[pl.ds(0, 128)], g0).wait()
                pltpu.sync_copy(rows0.at[pl.ds(0, 128)],
                                out_hbm.at[pl.ds((b0 + b) * 128, 128)])

    return k(table, idx2d)


def _sc_scatter_add(rows, idx2d, init):
    NB = idx2d.shape[0]
    nper = NB // 16
    nmain = nper // (2 * KS)
    W = KS * 128

    @functools.partial(
        pl.kernel,
        out_type=jax.ShapeDtypeStruct((N1P, U), jnp.float32),
        mesh=plsc.VectorSubcoreMesh(**_MESH),
        scratch_types=[
            pltpu.VMEM((KS, 128), jnp.int32),
            pltpu.VMEM((KS, 128), jnp.int32),
            pltpu.VMEM((W, CW), jnp.float32),
            pltpu.VMEM((W, CW), jnp.float32),
            pltpu.VMEM_SHARED((N1P, CW), jnp.float32),
            pltpu.SemaphoreType.DMA,
            pltpu.SemaphoreType.DMA,
            pltpu.SemaphoreType.DMA,
            pltpu.SemaphoreType.DMA,
        ],
        compiler_params=pltpu.CompilerParams(use_tc_tiling_on_sc=False),
    )
    def k(rows_hbm, idx_hbm, init_hbm, acc_hbm,
          idx0, idx1, rows0, rows1, acc_sh, c0, c1, a0, a1):
        c = lax.axis_index("c")
        s = lax.axis_index("s")
        idxs_v = (idx0, idx1)
        rows_v = (rows0, rows1)
        csem = (c0, c1)
        asem = (a0, a1)
        b0 = s * nper
        for p in range(2):
            col0 = (c * 2 + p) * CW
            pltpu.sync_copy(
                init_hbm.at[pl.ds(s * ROWS_PER_TILE, ROWS_PER_TILE), pl.ds(col0, CW)],
                acc_sh.at[pl.ds(s * ROWS_PER_TILE, ROWS_PER_TILE)])
            plsc.subcore_barrier()

            @pl.loop(0, nmain)
            def _(j):
                ld = []
                for d in range(2):
                    bb = (j * 2 + d) * KS
                    ld.append((
                        pltpu.async_copy(idx_hbm.at[pl.ds(b0 + bb, KS)],
                                         idxs_v[d], csem[d]),
                        pltpu.async_copy(
                            rows_hbm.at[pl.ds((b0 + bb) * 128, W),
                                        pl.ds(col0, CW)],
                            rows_v[d], csem[d]),
                    ))
                adds = []
                for d in range(2):
                    for cp in ld[d]:
                        cp.wait()
                    adds.extend(
                        pltpu.async_copy(rows_v[d].at[pl.ds(kk * 128, 128)],
                                         acc_sh.at[idxs_v[d].at[kk]],
                                         asem[d], add=True)
                        for kk in range(KS))
                for cp in adds:
                    cp.wait()

            if nmain * 2 * KS < nper:
                @pl.loop(nmain * 2 * KS, nper)
                def _(b):
                    pltpu.sync_copy(idx_hbm.at[pl.ds(b0 + b, 1)],
                                    idx0.at[pl.ds(0, 1)])
                    pltpu.sync_copy(
                        rows_hbm.at[pl.ds((b0 + b) * 128, 128), pl.ds(col0, CW)],
                        rows0.at[pl.ds(0, 128)])
                    pltpu.async_copy(rows0.at[pl.ds(0, 128)],
                                     acc_sh.at[idx0.at[0]], a0, add=True).wait()

            plsc.subcore_barrier()
            pltpu.sync_copy(
                acc_sh.at[pl.ds(s * ROWS_PER_TILE, ROWS_PER_TILE)],
                acc_hbm.at[pl.ds(s * ROWS_PER_TILE, ROWS_PER_TILE), pl.ds(col0, CW)],
            )
            plsc.subcore_barrier()

    return k(rows, idx2d, init)


_R = 512


def _tc_fin(h0p, w, b):
    def body(x_ref, w_ref, b_ref, o_ref):
        o_ref[...] = jnp.tanh(_dot(x_ref[...], w_ref[...]) + b_ref[...])

    return pl.pallas_call(
        body,
        grid=(N1P // _R,),
        in_specs=[
            pl.BlockSpec((_R, U), lambda i: (i, 0)),
            pl.BlockSpec((U, U), lambda i: (0, 0)),
            pl.BlockSpec((1, U), lambda i: (0, 0)),
        ],
        out_specs=pl.BlockSpec((_R, U), lambda i: (i, 0)),
        out_shape=jax.ShapeDtypeStruct((N1P, U), jnp.float32),
    )(h0p, w, b)


def _tc_gru(m, wih, whh, bih, bhh, wd1g, T, ngp, want_hg):
    def body(m_ref, wih_ref, whh_ref, bih_ref, bhh_ref, wd1_ref, y_ref,
             *maybe_hg):
        wih_v = wih_ref[...]
        whh_v = whh_ref[...]
        bih_v = bih_ref[...]
        bhh_v = bhh_ref[...]
        wd1_v = wd1_ref[...]
        h = None
        for t in range(T):
            gi = _dot(m_ref[t], wih_v) + bih_v
            gh = bhh_v if h is None else _dot(h, whh_v) + bhh_v
            r = jax.nn.sigmoid(gi[:, 0:U] + gh[..., 0:U])
            z = jax.nn.sigmoid(gi[:, U:2 * U] + gh[..., U:2 * U])
            n = jnp.tanh(gi[:, 2 * U:] + r * gh[..., 2 * U:])
            h = n - z * n if h is None else (1.0 - z) * n + z * h
            y_ref[t] = _dot(h, wd1_v)
        if want_hg:
            maybe_hg[0][...] = h

    out_shape = [jax.ShapeDtypeStruct((T, ngp, U), jnp.float32)]
    out_specs = [pl.BlockSpec((T, _R, U), lambda i: (0, i, 0))]
    if want_hg:
        out_shape.append(jax.ShapeDtypeStruct((ngp, U), jnp.float32))
        out_specs.append(pl.BlockSpec((_R, U), lambda i: (i, 0)))
    return pl.pallas_call(
        body,
        grid=(ngp // _R,),
        in_specs=[
            pl.BlockSpec((T, _R, U), lambda i: (0, i, 0)),
            pl.BlockSpec((U, 3 * U), lambda i: (0, 0)),
            pl.BlockSpec((U, 3 * U), lambda i: (0, 0)),
            pl.BlockSpec((1, 3 * U), lambda i: (0, 0)),
            pl.BlockSpec((1, 3 * U), lambda i: (0, 0)),
            pl.BlockSpec((U, U), lambda i: (0, 0)),
        ],
        out_specs=out_specs,
        out_shape=out_shape,
    )(m, wih, whh, bih, bhh, wd1g)


def _tc_combine(h, acc, w_h, bd1, wd2, bd2):
    def body(h_ref, a_ref, wh_ref, b1_ref, wd2_ref, b2_ref, o_ref):
        t = _dot(h_ref[...], wh_ref[...]) + a_ref[...] + b1_ref[...]
        o_ref[...] = jnp.tanh(_dot(jnp.tanh(t), wd2_ref[...]) + b2_ref[...])

    rspec = pl.BlockSpec((_R, U), lambda i: (i, 0))
    wspec = pl.BlockSpec((U, U), lambda i: (0, 0))
    bspec = pl.BlockSpec((1, U), lambda i: (0, 0))
    return pl.pallas_call(
        body,
        grid=(N1P // _R,),
        in_specs=[rspec, rspec, wspec, bspec, wspec, bspec],
        out_specs=rspec,
        out_shape=jax.ShapeDtypeStruct((N1P, U), jnp.float32),
    )(h, acc, w_h, bd1, wd2, bd2)


def _tc_readout(x3, t_idx, w1, b1, w2p, b2p):
    np_rows = x3.shape[1]

    def body(x_ref, w1_ref, b1_ref, w2_ref, b2_ref, o_ref):
        t = _dot(x_ref[0], w1_ref[...]) + b1_ref[...]
        o_ref[...] = _dot(t, w2_ref[...]) + b2_ref[...]

    return pl.pallas_call(
        body,
        grid=(np_rows // _R,),
        in_specs=[
            pl.BlockSpec((1, _R, U), lambda i: (t_idx, i, 0)),
            pl.BlockSpec((U, U), lambda i: (0, 0)),
            pl.BlockSpec((1, U), lambda i: (0, 0)),
            pl.BlockSpec((U, 8), lambda i: (0, 0)),
            pl.BlockSpec((1, 8), lambda i: (0, 0)),
        ],
        out_specs=pl.BlockSpec((_R, 8), lambda i: (i, 0)),
        out_shape=jax.ShapeDtypeStruct((np_rows, 8), jnp.float32),
    )(x3, w1, b1, w2p, b2p)


def _layer(h, L, p, idx2ds, zeros_acc, want_hg):
    wd1 = p[L + "_Wd1"]
    hgs = {}
    acc = zeros_acc
    for gi, (name, (T, _, ngp)) in enumerate(GDEFS.items()):
        idx2d = idx2ds[name]
        m = _sc_gather(h, idx2d).reshape(T, ngp, U)
        outs = _tc_gru(m, p[L + "_Wih"], p[L + "_Whh"],
                       p[L + "_bih"][None, :], p[L + "_bhh"][None, :],
                       wd1[U * (gi + 1):U * (gi + 2)], T, ngp, want_hg)
        if want_hg:
            hgs[name] = outs[1]
        acc = _sc_scatter_add(outs[0].reshape(T * ngp, U), idx2d, acc)
    hnew = _tc_combine(h, acc, wd1[0:U],
                       p[L + "_bd1"][None, :], p[L + "_Wd2"], p[L + "_bd2"][None, :])
    return hnew, hgs


def kernel(h0, params, g2_idx, g3_idx, g4_idx):
    p = params
    idxs = {"g2": g2_idx, "g3": g3_idx, "g4": g4_idx}

    idx2ds = {}
    for name, (T, ng, ngp) in GDEFS.items():
        it = jnp.full((T, ngp), DUMP, jnp.int32)
        it = it.at[:, :ng].set(idxs[name].astype(jnp.int32).T)
        idx2ds[name] = it.reshape(-1, 128)

    h0p = jnp.pad(h0, ((0, N1P - N1), (0, U - h0.shape[1])))
    finw = jnp.pad(p["fin_W"], ((0, U - p["fin_W"].shape[0]), (0, 0)))
    zeros_acc = jnp.zeros((N1P, U), jnp.float32)

    h = _tc_fin(h0p, finw, p["fin_b"][None, :])
    h, _ = _layer(h, "d0", p, idx2ds, zeros_acc, False)
    h, hgs = _layer(h, "d2", p, idx2ds, zeros_acc, True)

    outs = []
    ro_in = {
        "atom": (h, N1),
        "bond": (hgs["g2"], N1),
        "angle": (hgs["g3"], GDEFS["g3"][1]),
        "torsion": (hgs["g4"], GDEFS["g4"][1]),
    }
    for term, (x2, nreal) in ro_in.items():
        w2p = jnp.pad(p["fr_" + term + "_W2"], ((0, 0), (0, 6)))
        b2p = jnp.pad(p["fr_" + term + "_b2"], ((0, 6)))[None, :]
        o = _tc_readout(x2[None], 0, p["fr_" + term + "_W1"],
                        p["fr_" + term + "_b1"][None, :], w2p, b2p)
        outs.append(o[:nreal, :2])
    return jnp.concatenate(outs, axis=0)


# speedup vs baseline: 1.7739x; 1.3825x over previous
import functools

import jax
import jax.numpy as jnp
from jax import lax
from jax.experimental import pallas as pl
from jax.experimental.pallas import tpu as pltpu
from jax.experimental.pallas import tpu_sc as plsc

N1 = 50000
N1P = 51200
DUMP = 50000
U = 128
NCOL = 4
CW = U // NCOL
ROWS_PER_TILE = N1P // 16

GDEFS = {"g2": (2, 50000, 51200), "g3": (3, 80000, 81920), "g4": (4, 100000, 100352)}
KG = 2
KS = 1

_MESH = dict(core_axis_name="c", subcore_axis_name="s")
_BF = jnp.bfloat16


def _dot(a, b):
    return jnp.dot(a.astype(_BF), b.astype(_BF),
                   preferred_element_type=jnp.float32)


def _sc_gather(table, idx2d):
    NB = idx2d.shape[0]
    nper = NB // 32
    nmain = nper // (2 * KG)
    ntail0 = nmain * 2 * KG
    W = KG * 128

    @functools.partial(
        pl.kernel,
        out_type=jax.ShapeDtypeStruct((NB * 128, U), jnp.float32),
        mesh=plsc.VectorSubcoreMesh(**_MESH),
        scratch_types=[
            pltpu.VMEM((nper, 128), jnp.int32),
            pltpu.VMEM((W, U), jnp.float32),
            pltpu.VMEM((W, U), jnp.float32),
            pltpu.SemaphoreType.DMA,
            pltpu.SemaphoreType.DMA,
            pltpu.SemaphoreType.DMA,
            pltpu.SemaphoreType.DMA,
        ],
        compiler_params=pltpu.CompilerParams(use_tc_tiling_on_sc=False),
    )
    def k(table_hbm, idx_hbm, out_hbm, idx_v, rows0, rows1, g0, g1, o0, o1):
        wid = lax.axis_index("s") * 2 + lax.axis_index("c")
        b0 = wid * nper
        pltpu.sync_copy(idx_hbm.at[pl.ds(b0, nper)], idx_v)
        rows = (rows0, rows1)
        gsem = (g0, g1)
        osem = (o0, o1)

        @pl.loop(0, nmain)
        def _(jo):
            for d in range(2):
                base = (jo * 2 + d) * KG

                @pl.when(jo > 0)
                def _():
                    pltpu.make_async_copy(
                        rows[d], out_hbm.at[pl.ds(b0 * 128, W)], osem[d]).wait()

                for kk in range(KG):
                    pltpu.async_copy(table_hbm.at[idx_v.at[base + kk]],
                                     rows[d].at[pl.ds(kk * 128, 128)], gsem[d])
            for d in range(2):
                base = (jo * 2 + d) * KG
                for kk in range(KG):
                    pltpu.make_async_copy(
                        table_hbm.at[idx_v.at[kk]],
                        rows[d].at[pl.ds(kk * 128, 128)], gsem[d]).wait()
                pltpu.async_copy(rows[d],
                                 out_hbm.at[pl.ds((b0 + base) * 128, W)], osem[d])

        if nmain > 0:
            for d in range(2):
                pltpu.make_async_copy(
                    rows[d], out_hbm.at[pl.ds(b0 * 128, W)], osem[d]).wait()

        if ntail0 < nper:
            @pl.loop(ntail0, nper)
            def _(b):
                pltpu.async_copy(table_hbm.at[idx_v.at[b]],
                                 rows0.at[pl.ds(0, 128)], g0).wait()
                pltpu.sync_copy(rows0.at[pl.ds(0, 128)],
                                out_hbm.at[pl.ds((b0 + b) * 128, 128)])

    return k(table, idx2d)


def _sc_scatter_add(rows3, idx2d, zeros_tile):
    nbs = [r.shape[0] // 128 for r in rows3]
    assert sum(nbs) == idx2d.shape[0]
    W = KS * 128

    @functools.partial(
        pl.kernel,
        out_type=jax.ShapeDtypeStruct((N1P, U), jnp.float32),
        mesh=plsc.VectorSubcoreMesh(**_MESH),
        scratch_types=[
            pltpu.VMEM((KS, 128), jnp.int32),
            pltpu.VMEM((KS, 128), jnp.int32),
            pltpu.VMEM((W, CW), jnp.float32),
            pltpu.VMEM((W, CW), jnp.float32),
            pltpu.VMEM_SHARED((N1P, CW), jnp.float32),
            pltpu.SemaphoreType.DMA,
            pltpu.SemaphoreType.DMA,
            pltpu.SemaphoreType.DMA,
            pltpu.SemaphoreType.DMA,
        ],
        compiler_params=pltpu.CompilerParams(use_tc_tiling_on_sc=False),
    )
    def k(r2_hbm, r3_hbm, r4_hbm, idx_hbm, zeros_hbm, acc_hbm,
          idx0, idx1, rows0, rows1, acc_sh, c0, c1, a0, a1):
        c = lax.axis_index("c")
        s = lax.axis_index("s")
        idxs_v = (idx0, idx1)
        rows_v = (rows0, rows1)
        csem = (c0, c1)
        asem = (a0, a1)
        for p in range(2):
            col0 = (c * 2 + p) * CW
            pltpu.sync_copy(zeros_hbm,
                            acc_sh.at[pl.ds(s * ROWS_PER_TILE, ROWS_PER_TILE)])
            plsc.subcore_barrier()

            nboff = 0
            for rows_hbm, nb in zip((r2_hbm, r3_hbm, r4_hbm), nbs):
                nper = nb // 16
                nmain = nper // (2 * KS)
                b0 = nboff + s * nper
                r0 = s * nper

                @pl.loop(0, nmain)
                def _(j):
                    ld = []
                    for d in range(2):
                        bb = (j * 2 + d) * KS
                        ld.append((
                            pltpu.async_copy(idx_hbm.at[pl.ds(b0 + bb, KS)],
                                             idxs_v[d], csem[d]),
                            pltpu.async_copy(
                                rows_hbm.at[pl.ds((r0 + bb) * 128, W),
                                            pl.ds(col0, CW)],
                                rows_v[d], csem[d]),
                        ))
                    adds = []
                    for d in range(2):
                        for cp in ld[d]:
                            cp.wait()
                        adds.extend(
                            pltpu.async_copy(rows_v[d].at[pl.ds(kk * 128, 128)],
                                             acc_sh.at[idxs_v[d].at[kk]],
                                             asem[d], add=True)
                            for kk in range(KS))
                    for cp in adds:
                        cp.wait()

                if nmain * 2 * KS < nper:
                    @pl.loop(nmain * 2 * KS, nper)
                    def _(b):
                        pltpu.sync_copy(idx_hbm.at[pl.ds(b0 + b, 1)],
                                        idx0.at[pl.ds(0, 1)])
                        pltpu.sync_copy(
                            rows_hbm.at[pl.ds((r0 + b) * 128, 128),
                                        pl.ds(col0, CW)],
                            rows0.at[pl.ds(0, 128)])
                        pltpu.async_copy(rows0.at[pl.ds(0, 128)],
                                         acc_sh.at[idx0.at[0]], a0,
                                         add=True).wait()
                nboff += nb

            plsc.subcore_barrier()
            pltpu.sync_copy(
                acc_sh.at[pl.ds(s * ROWS_PER_TILE, ROWS_PER_TILE)],
                acc_hbm.at[pl.ds(s * ROWS_PER_TILE, ROWS_PER_TILE),
                           pl.ds(col0, CW)],
            )
            plsc.subcore_barrier()

    return k(rows3[0], rows3[1], rows3[2], idx2d, zeros_tile)


_R = 512


def _tc_fin(h0p, w, b):
    def body(x_ref, w_ref, b_ref, o_ref):
        o_ref[...] = jnp.tanh(_dot(x_ref[...], w_ref[...]) + b_ref[...])

    return pl.pallas_call(
        body,
        grid=(N1P // _R,),
        in_specs=[
            pl.BlockSpec((_R, U), lambda i: (i, 0)),
            pl.BlockSpec((U, U), lambda i: (0, 0)),
            pl.BlockSpec((1, U), lambda i: (0, 0)),
        ],
        out_specs=pl.BlockSpec((_R, U), lambda i: (i, 0)),
        out_shape=jax.ShapeDtypeStruct((N1P, U), jnp.float32),
    )(h0p, w, b)


def _tc_gru(m, wih, whh, bih, bhh, wd1g, T, ngp, want_hg):
    def body(m_ref, wih_ref, whh_ref, bih_ref, bhh_ref, wd1_ref, y_ref,
             *maybe_hg):
        wih_v = wih_ref[...]
        whh_v = whh_ref[...]
        bih_v = bih_ref[...]
        bhh_v = bhh_ref[...]
        wd1_v = wd1_ref[...]
        h = None
        for t in range(T):
            gi = _dot(m_ref[t], wih_v) + bih_v
            gh = bhh_v if h is None else _dot(h, whh_v) + bhh_v
            r = jax.nn.sigmoid(gi[:, 0:U] + gh[..., 0:U])
            z = jax.nn.sigmoid(gi[:, U:2 * U] + gh[..., U:2 * U])
            n = jnp.tanh(gi[:, 2 * U:] + r * gh[..., 2 * U:])
            h = n - z * n if h is None else (1.0 - z) * n + z * h
            y_ref[t] = _dot(h, wd1_v)
        if want_hg:
            maybe_hg[0][...] = h

    out_shape = [jax.ShapeDtypeStruct((T, ngp, U), jnp.float32)]
    out_specs = [pl.BlockSpec((T, _R, U), lambda i: (0, i, 0))]
    if want_hg:
        out_shape.append(jax.ShapeDtypeStruct((ngp, U), jnp.float32))
        out_specs.append(pl.BlockSpec((_R, U), lambda i: (i, 0)))
    return pl.pallas_call(
        body,
        grid=(ngp // _R,),
        in_specs=[
            pl.BlockSpec((T, _R, U), lambda i: (0, i, 0)),
            pl.BlockSpec((U, 3 * U), lambda i: (0, 0)),
            pl.BlockSpec((U, 3 * U), lambda i: (0, 0)),
            pl.BlockSpec((1, 3 * U), lambda i: (0, 0)),
            pl.BlockSpec((1, 3 * U), lambda i: (0, 0)),
            pl.BlockSpec((U, U), lambda i: (0, 0)),
        ],
        out_specs=out_specs,
        out_shape=out_shape,
    )(m, wih, whh, bih, bhh, wd1g)


def _tc_combine(h, acc, w_h, bd1, wd2, bd2):
    def body(h_ref, a_ref, wh_ref, b1_ref, wd2_ref, b2_ref, o_ref):
        t = _dot(h_ref[...], wh_ref[...]) + a_ref[...] + b1_ref[...]
        o_ref[...] = jnp.tanh(_dot(jnp.tanh(t), wd2_ref[...]) + b2_ref[...])

    rspec = pl.BlockSpec((_R, U), lambda i: (i, 0))
    wspec = pl.BlockSpec((U, U), lambda i: (0, 0))
    bspec = pl.BlockSpec((1, U), lambda i: (0, 0))
    return pl.pallas_call(
        body,
        grid=(N1P // _R,),
        in_specs=[rspec, rspec, wspec, bspec, wspec, bspec],
        out_specs=rspec,
        out_shape=jax.ShapeDtypeStruct((N1P, U), jnp.float32),
    )(h, acc, w_h, bd1, wd2, bd2)


def _tc_readout(x3, t_idx, w1, b1, w2p, b2p):
    np_rows = x3.shape[1]

    def body(x_ref, w1_ref, b1_ref, w2_ref, b2_ref, o_ref):
        t = _dot(x_ref[0], w1_ref[...]) + b1_ref[...]
        o_ref[...] = _dot(t, w2_ref[...]) + b2_ref[...]

    return pl.pallas_call(
        body,
        grid=(np_rows // _R,),
        in_specs=[
            pl.BlockSpec((1, _R, U), lambda i: (t_idx, i, 0)),
            pl.BlockSpec((U, U), lambda i: (0, 0)),
            pl.BlockSpec((1, U), lambda i: (0, 0)),
            pl.BlockSpec((U, 8), lambda i: (0, 0)),
            pl.BlockSpec((1, 8), lambda i: (0, 0)),
        ],
        out_specs=pl.BlockSpec((_R, 8), lambda i: (i, 0)),
        out_shape=jax.ShapeDtypeStruct((np_rows, 8), jnp.float32),
    )(x3, w1, b1, w2p, b2p)


def _layer(h, L, p, idx_all, zeros_tile, want_hg):
    wd1 = p[L + "_Wd1"]
    m_all = _sc_gather(h, idx_all)
    hgs = {}
    ys = []
    roff = 0
    for gi, (name, (T, _, ngp)) in enumerate(GDEFS.items()):
        m = m_all[roff:roff + T * ngp].reshape(T, ngp, U)
        roff += T * ngp
        outs = _tc_gru(m, p[L + "_Wih"], p[L + "_Whh"],
                       p[L + "_bih"][None, :], p[L + "_bhh"][None, :],
                       wd1[U * (gi + 1):U * (gi + 2)], T, ngp, want_hg)
        ys.append(outs[0].reshape(T * ngp, U))
        if want_hg:
            hgs[name] = outs[1]
    acc = _sc_scatter_add(ys, idx_all, zeros_tile)
    hnew = _tc_combine(h, acc, wd1[0:U],
                       p[L + "_bd1"][None, :], p[L + "_Wd2"], p[L + "_bd2"][None, :])
    return hnew, hgs


def kernel(h0, params, g2_idx, g3_idx, g4_idx):
    p = params
    idxs = {"g2": g2_idx, "g3": g3_idx, "g4": g4_idx}

    idx2ds = []
    for name, (T, ng, ngp) in GDEFS.items():
        it = jnp.full((T, ngp), DUMP, jnp.int32)
        it = it.at[:, :ng].set(idxs[name].astype(jnp.int32).T)
        idx2ds.append(it.reshape(-1, 128))
    idx_all = jnp.concatenate(idx2ds, axis=0)

    h0p = jnp.pad(h0, ((0, N1P - N1), (0, U - h0.shape[1])))
    finw = jnp.pad(p["fin_W"], ((0, U - p["fin_W"].shape[0]), (0, 0)))
    zeros_tile = jnp.zeros((ROWS_PER_TILE, CW), jnp.float32)

    h = _tc_fin(h0p, finw, p["fin_b"][None, :])
    h, _ = _layer(h, "d0", p, idx_all, zeros_tile, False)
    h, hgs = _layer(h, "d2", p, idx_all, zeros_tile, True)

    outs = []
    ro_in = {
        "atom": (h, N1),
        "bond": (hgs["g2"], N1),
        "angle": (hgs["g3"], GDEFS["g3"][1]),
        "torsion": (hgs["g4"], GDEFS["g4"][1]),
    }
    for term, (x2, nreal) in ro_in.items():
        w2p = jnp.pad(p["fr_" + term + "_W2"], ((0, 0), (0, 6)))
        b2p = jnp.pad(p["fr_" + term + "_b2"], ((0, 6)))[None, :]
        o = _tc_readout(x2[None], 0, p["fr_" + term + "_W1"],
                        p["fr_" + term + "_b1"][None, :], w2p, b2p)
        outs.append(o[:nreal, :2])
    return jnp.concatenate(outs, axis=0)


# speedup vs baseline: 1.9722x; 1.1118x over previous
import functools

import jax
import jax.numpy as jnp
from jax import lax
from jax.experimental import pallas as pl
from jax.experimental.pallas import tpu as pltpu
from jax.experimental.pallas import tpu_sc as plsc

N1 = 50000
N1P = 51200
DUMP = 50000
U = 128
NCOL = 4
CW = U // NCOL
ROWS_PER_TILE = N1P // 16

GDEFS = {"g2": (2, 50000, 51200), "g3": (3, 80000, 81920), "g4": (4, 100000, 100352)}
KG = 2
KS = 1

_MESH = dict(core_axis_name="c", subcore_axis_name="s")
_BF = jnp.bfloat16


def _dot(a, b):
    return jnp.dot(a.astype(_BF), b.astype(_BF),
                   preferred_element_type=jnp.float32)


def _sc_gather(table, idx2d, nbs):
    assert sum(nbs) == idx2d.shape[0]
    npers = [nb // 32 for nb in nbs]
    W = KG * 128

    @functools.partial(
        pl.kernel,
        out_type=[jax.ShapeDtypeStruct((nb * 128, U), jnp.float32)
                  for nb in nbs],
        mesh=plsc.VectorSubcoreMesh(**_MESH),
        scratch_types=[
            pltpu.VMEM((max(npers), 128), jnp.int32),
            pltpu.VMEM((W, U), jnp.float32),
            pltpu.VMEM((W, U), jnp.float32),
            pltpu.SemaphoreType.DMA,
            pltpu.SemaphoreType.DMA,
            pltpu.SemaphoreType.DMA,
            pltpu.SemaphoreType.DMA,
        ],
        compiler_params=pltpu.CompilerParams(use_tc_tiling_on_sc=False),
    )
    def k(table_hbm, idx_hbm, o2, o3, o4, idx_v, rows0, rows1, g0, g1, o0, o1):
        wid = lax.axis_index("s") * 2 + lax.axis_index("c")
        rows = (rows0, rows1)
        gsem = (g0, g1)
        osem = (o0, o1)

        nboff = 0
        for out_hbm, nper in zip((o2, o3, o4), npers):
            nmain = nper // (2 * KG)
            ntail0 = nmain * 2 * KG
            b0 = wid * nper
            pltpu.sync_copy(idx_hbm.at[pl.ds(nboff + b0, nper)],
                            idx_v.at[pl.ds(0, nper)])

            @pl.loop(0, nmain)
            def _(jo):
                for d in range(2):
                    base = (jo * 2 + d) * KG

                    @pl.when(jo > 0)
                    def _():
                        pltpu.make_async_copy(
                            rows[d], out_hbm.at[pl.ds(b0 * 128, W)],
                            osem[d]).wait()

                    for kk in range(KG):
                        pltpu.async_copy(table_hbm.at[idx_v.at[base + kk]],
                                         rows[d].at[pl.ds(kk * 128, 128)],
                                         gsem[d])
                for d in range(2):
                    base = (jo * 2 + d) * KG
                    for kk in range(KG):
                        pltpu.make_async_copy(
                            table_hbm.at[idx_v.at[kk]],
                            rows[d].at[pl.ds(kk * 128, 128)], gsem[d]).wait()
                    pltpu.async_copy(
                        rows[d], out_hbm.at[pl.ds((b0 + base) * 128, W)],
                        osem[d])

            if nmain > 0:
                for d in range(2):
                    pltpu.make_async_copy(
                        rows[d], out_hbm.at[pl.ds(b0 * 128, W)], osem[d]).wait()

            if ntail0 < nper:
                @pl.loop(ntail0, nper)
                def _(b):
                    pltpu.async_copy(table_hbm.at[idx_v.at[b]],
                                     rows0.at[pl.ds(0, 128)], g0).wait()
                    pltpu.sync_copy(rows0.at[pl.ds(0, 128)],
                                    out_hbm.at[pl.ds((b0 + b) * 128, 128)])
            nboff += nper * 32

    return k(table, idx2d)


def _sc_scatter_add(rows3, idx2d, zeros_tile):
    nbs = [r.shape[0] // 128 for r in rows3]
    assert sum(nbs) == idx2d.shape[0]
    W = KS * 128

    @functools.partial(
        pl.kernel,
        out_type=jax.ShapeDtypeStruct((N1P, U), jnp.float32),
        mesh=plsc.VectorSubcoreMesh(**_MESH),
        scratch_types=[
            pltpu.VMEM((KS, 128), jnp.int32),
            pltpu.VMEM((KS, 128), jnp.int32),
            pltpu.VMEM((W, CW), jnp.float32),
            pltpu.VMEM((W, CW), jnp.float32),
            pltpu.VMEM_SHARED((N1P, CW), jnp.float32),
            pltpu.SemaphoreType.DMA,
            pltpu.SemaphoreType.DMA,
            pltpu.SemaphoreType.DMA,
            pltpu.SemaphoreType.DMA,
        ],
        compiler_params=pltpu.CompilerParams(use_tc_tiling_on_sc=False),
    )
    def k(r2_hbm, r3_hbm, r4_hbm, idx_hbm, zeros_hbm, acc_hbm,
          idx0, idx1, rows0, rows1, acc_sh, c0, c1, a0, a1):
        c = lax.axis_index("c")
        s = lax.axis_index("s")
        idxs_v = (idx0, idx1)
        rows_v = (rows0, rows1)
        csem = (c0, c1)
        asem = (a0, a1)
        for p in range(2):
            col0 = (c * 2 + p) * CW
            pltpu.sync_copy(zeros_hbm,
                            acc_sh.at[pl.ds(s * ROWS_PER_TILE, ROWS_PER_TILE)])
            plsc.subcore_barrier()

            nboff = 0
            for rows_hbm, nb in zip((r2_hbm, r3_hbm, r4_hbm), nbs):
                nper = nb // 16
                nmain = nper // (2 * KS)
                b0 = nboff + s * nper
                r0 = s * nper

                @pl.loop(0, nmain)
                def _(j):
                    ld = []
                    for d in range(2):
                        bb = (j * 2 + d) * KS
                        ld.append((
                            pltpu.async_copy(idx_hbm.at[pl.ds(b0 + bb, KS)],
                                             idxs_v[d], csem[d]),
                            pltpu.async_copy(
                                rows_hbm.at[pl.ds((r0 + bb) * 128, W),
                                            pl.ds(col0, CW)],
                                rows_v[d], csem[d]),
                        ))
                    adds = []
                    for d in range(2):
                        for cp in ld[d]:
                            cp.wait()
                        adds.extend(
                            pltpu.async_copy(rows_v[d].at[pl.ds(kk * 128, 128)],
                                             acc_sh.at[idxs_v[d].at[kk]],
                                             asem[d], add=True)
                            for kk in range(KS))
                    for cp in adds:
                        cp.wait()

                if nmain * 2 * KS < nper:
                    @pl.loop(nmain * 2 * KS, nper)
                    def _(b):
                        pltpu.sync_copy(idx_hbm.at[pl.ds(b0 + b, 1)],
                                        idx0.at[pl.ds(0, 1)])
                        pltpu.sync_copy(
                            rows_hbm.at[pl.ds((r0 + b) * 128, 128),
                                        pl.ds(col0, CW)],
                            rows0.at[pl.ds(0, 128)])
                        pltpu.async_copy(rows0.at[pl.ds(0, 128)],
                                         acc_sh.at[idx0.at[0]], a0,
                                         add=True).wait()
                nboff += nb

            plsc.subcore_barrier()
            pltpu.sync_copy(
                acc_sh.at[pl.ds(s * ROWS_PER_TILE, ROWS_PER_TILE)],
                acc_hbm.at[pl.ds(s * ROWS_PER_TILE, ROWS_PER_TILE),
                           pl.ds(col0, CW)],
            )
            plsc.subcore_barrier()

    return k(rows3[0], rows3[1], rows3[2], idx2d, zeros_tile)


_R = 512


def _tc_fin(h0p, w, b):
    def body(x_ref, w_ref, b_ref, o_ref):
        o_ref[...] = jnp.tanh(_dot(x_ref[...], w_ref[...]) + b_ref[...])

    return pl.pallas_call(
        body,
        grid=(N1P // _R,),
        in_specs=[
            pl.BlockSpec((_R, U), lambda i: (i, 0)),
            pl.BlockSpec((U, U), lambda i: (0, 0)),
            pl.BlockSpec((1, U), lambda i: (0, 0)),
        ],
        out_specs=pl.BlockSpec((_R, U), lambda i: (i, 0)),
        out_shape=jax.ShapeDtypeStruct((N1P, U), jnp.float32),
    )(h0p, w, b)


def _tc_gru(m, wih, whh, bih, bhh, wd1g, T, ngp, want_hg):
    def body(m_ref, wih_ref, whh_ref, bih_ref, bhh_ref, wd1_ref, y_ref,
             *maybe_hg):
        wih_v = wih_ref[...]
        whh_v = whh_ref[...]
        bih_v = bih_ref[...]
        bhh_v = bhh_ref[...]
        wd1_v = wd1_ref[...]
        h = None
        for t in range(T):
            gi = _dot(m_ref[t], wih_v) + bih_v
            gh = bhh_v if h is None else _dot(h, whh_v) + bhh_v
            r = jax.nn.sigmoid(gi[:, 0:U] + gh[..., 0:U])
            z = jax.nn.sigmoid(gi[:, U:2 * U] + gh[..., U:2 * U])
            n = jnp.tanh(gi[:, 2 * U:] + r * gh[..., 2 * U:])
            h = n - z * n if h is None else (1.0 - z) * n + z * h
            y_ref[t] = _dot(h, wd1_v)
        if want_hg:
            maybe_hg[0][...] = h

    out_shape = [jax.ShapeDtypeStruct((T, ngp, U), jnp.float32)]
    out_specs = [pl.BlockSpec((T, _R, U), lambda i: (0, i, 0))]
    if want_hg:
        out_shape.append(jax.ShapeDtypeStruct((ngp, U), jnp.float32))
        out_specs.append(pl.BlockSpec((_R, U), lambda i: (i, 0)))
    return pl.pallas_call(
        body,
        grid=(ngp // _R,),
        in_specs=[
            pl.BlockSpec((T, _R, U), lambda i: (0, i, 0)),
            pl.BlockSpec((U, 3 * U), lambda i: (0, 0)),
            pl.BlockSpec((U, 3 * U), lambda i: (0, 0)),
            pl.BlockSpec((1, 3 * U), lambda i: (0, 0)),
            pl.BlockSpec((1, 3 * U), lambda i: (0, 0)),
            pl.BlockSpec((U, U), lambda i: (0, 0)),
        ],
        out_specs=out_specs,
        out_shape=out_shape,
    )(m, wih, whh, bih, bhh, wd1g)


def _tc_combine(h, acc, w_h, bd1, wd2, bd2):
    def body(h_ref, a_ref, wh_ref, b1_ref, wd2_ref, b2_ref, o_ref):
        t = _dot(h_ref[...], wh_ref[...]) + a_ref[...] + b1_ref[...]
        o_ref[...] = jnp.tanh(_dot(jnp.tanh(t), wd2_ref[...]) + b2_ref[...])

    rspec = pl.BlockSpec((_R, U), lambda i: (i, 0))
    wspec = pl.BlockSpec((U, U), lambda i: (0, 0))
    bspec = pl.BlockSpec((1, U), lambda i: (0, 0))
    return pl.pallas_call(
        body,
        grid=(N1P // _R,),
        in_specs=[rspec, rspec, wspec, bspec, wspec, bspec],
        out_specs=rspec,
        out_shape=jax.ShapeDtypeStruct((N1P, U), jnp.float32),
    )(h, acc, w_h, bd1, wd2, bd2)


def _tc_readout(x3, t_idx, w1, b1, w2p, b2p):
    np_rows = x3.shape[1]

    def body(x_ref, w1_ref, b1_ref, w2_ref, b2_ref, o_ref):
        t = _dot(x_ref[0], w1_ref[...]) + b1_ref[...]
        o_ref[...] = _dot(t, w2_ref[...]) + b2_ref[...]

    return pl.pallas_call(
        body,
        grid=(np_rows // _R,),
        in_specs=[
            pl.BlockSpec((1, _R, U), lambda i: (t_idx, i, 0)),
            pl.BlockSpec((U, U), lambda i: (0, 0)),
            pl.BlockSpec((1, U), lambda i: (0, 0)),
            pl.BlockSpec((U, 8), lambda i: (0, 0)),
            pl.BlockSpec((1, 8), lambda i: (0, 0)),
        ],
        out_specs=pl.BlockSpec((_R, 8), lambda i: (i, 0)),
        out_shape=jax.ShapeDtypeStruct((np_rows, 8), jnp.float32),
    )(x3, w1, b1, w2p, b2p)


def _layer(h, L, p, idx_all, zeros_tile, want_hg):
    wd1 = p[L + "_Wd1"]
    nbs = [T * ngp // 128 for (T, _, ngp) in GDEFS.values()]
    ms = _sc_gather(h, idx_all, nbs)
    hgs = {}
    ys = []
    for gi, (name, (T, _, ngp)) in enumerate(GDEFS.items()):
        m = ms[gi].reshape(T, ngp, U)
        outs = _tc_gru(m, p[L + "_Wih"], p[L + "_Whh"],
                       p[L + "_bih"][None, :], p[L + "_bhh"][None, :],
                       wd1[U * (gi + 1):U * (gi + 2)], T, ngp, want_hg)
        ys.append(outs[0].reshape(T * ngp, U))
        if want_hg:
            hgs[name] = outs[1]
    acc = _sc_scatter_add(ys, idx_all, zeros_tile)
    hnew = _tc_combine(h, acc, wd1[0:U],
                       p[L + "_bd1"][None, :], p[L + "_Wd2"], p[L + "_bd2"][None, :])
    return hnew, hgs


def kernel(h0, params, g2_idx, g3_idx, g4_idx):
    p = params
    idxs = {"g2": g2_idx, "g3": g3_idx, "g4": g4_idx}

    idx2ds = []
    for name, (T, ng, ngp) in GDEFS.items():
        it = jnp.full((T, ngp), DUMP, jnp.int32)
        it = it.at[:, :ng].set(idxs[name].astype(jnp.int32).T)
        idx2ds.append(it.reshape(-1, 128))
    idx_all = jnp.concatenate(idx2ds, axis=0)

    h0p = jnp.pad(h0, ((0, N1P - N1), (0, U - h0.shape[1])))
    finw = jnp.pad(p["fin_W"], ((0, U - p["fin_W"].shape[0]), (0, 0)))
    zeros_tile = jnp.zeros((ROWS_PER_TILE, CW), jnp.float32)

    h = _tc_fin(h0p, finw, p["fin_b"][None, :])
    h, _ = _layer(h, "d0", p, idx_all, zeros_tile, False)
    h, hgs = _layer(h, "d2", p, idx_all, zeros_tile, True)

    outs = []
    ro_in = {
        "atom": (h, N1),
        "bond": (hgs["g2"], N1),
        "angle": (hgs["g3"], GDEFS["g3"][1]),
        "torsion": (hgs["g4"], GDEFS["g4"][1]),
    }
    for term, (x2, nreal) in ro_in.items():
        w2p = jnp.pad(p["fr_" + term + "_W2"], ((0, 0), (0, 6)))
        b2p = jnp.pad(p["fr_" + term + "_b2"], ((0, 6)))[None, :]
        o = _tc_readout(x2[None], 0, p["fr_" + term + "_W1"],
                        p["fr_" + term + "_b1"][None, :], w2p, b2p)
        outs.append(o[:nreal, :2])
    return jnp.concatenate(outs, axis=0)


# speedup vs baseline: 1.9774x; 1.0027x over previous
import functools

import jax
import jax.numpy as jnp
from jax import lax
from jax.experimental import pallas as pl
from jax.experimental.pallas import tpu as pltpu
from jax.experimental.pallas import tpu_sc as plsc

N1 = 50000
N1P = 51200
DUMP = 50000
U = 128
NCOL = 4
CW = U // NCOL
ROWS_PER_TILE = N1P // 16

GDEFS = {"g2": (2, 50000, 51200), "g3": (3, 80000, 81920), "g4": (4, 100000, 100352)}
KG = 3
KS = 1

_MESH = dict(core_axis_name="c", subcore_axis_name="s")
_BF = jnp.bfloat16


def _dot(a, b):
    return jnp.dot(a.astype(_BF), b.astype(_BF),
                   preferred_element_type=jnp.float32)


def _sc_gather(table, idx2d, nbs):
    assert sum(nbs) == idx2d.shape[0]
    npers = [nb // 32 for nb in nbs]
    W = KG * 128

    @functools.partial(
        pl.kernel,
        out_type=[jax.ShapeDtypeStruct((nb * 128, U), jnp.float32)
                  for nb in nbs],
        mesh=plsc.VectorSubcoreMesh(**_MESH),
        scratch_types=[
            pltpu.VMEM((max(npers), 128), jnp.int32),
            pltpu.VMEM((W, U), jnp.float32),
            pltpu.VMEM((W, U), jnp.float32),
            pltpu.SemaphoreType.DMA,
            pltpu.SemaphoreType.DMA,
            pltpu.SemaphoreType.DMA,
            pltpu.SemaphoreType.DMA,
        ],
        compiler_params=pltpu.CompilerParams(use_tc_tiling_on_sc=False),
    )
    def k(table_hbm, idx_hbm, o2, o3, o4, idx_v, rows0, rows1, g0, g1, o0, o1):
        wid = lax.axis_index("s") * 2 + lax.axis_index("c")
        rows = (rows0, rows1)
        gsem = (g0, g1)
        osem = (o0, o1)

        nboff = 0
        for out_hbm, nper in zip((o2, o3, o4), npers):
            nmain = nper // (2 * KG)
            ntail0 = nmain * 2 * KG
            b0 = wid * nper
            pltpu.sync_copy(idx_hbm.at[pl.ds(nboff + b0, nper)],
                            idx_v.at[pl.ds(0, nper)])

            @pl.loop(0, nmain)
            def _(jo):
                for d in range(2):
                    base = (jo * 2 + d) * KG

                    @pl.when(jo > 0)
                    def _():
                        pltpu.make_async_copy(
                            rows[d], out_hbm.at[pl.ds(b0 * 128, W)],
                            osem[d]).wait()

                    for kk in range(KG):
                        pltpu.async_copy(table_hbm.at[idx_v.at[base + kk]],
                                         rows[d].at[pl.ds(kk * 128, 128)],
                                         gsem[d])
                for d in range(2):
                    base = (jo * 2 + d) * KG
                    for kk in range(KG):
                        pltpu.make_async_copy(
                            table_hbm.at[idx_v.at[kk]],
                            rows[d].at[pl.ds(kk * 128, 128)], gsem[d]).wait()
                    pltpu.async_copy(
                        rows[d], out_hbm.at[pl.ds((b0 + base) * 128, W)],
                        osem[d])

            if nmain > 0:
                for d in range(2):
                    pltpu.make_async_copy(
                        rows[d], out_hbm.at[pl.ds(b0 * 128, W)], osem[d]).wait()

            if ntail0 < nper:
                @pl.loop(ntail0, nper)
                def _(b):
                    pltpu.async_copy(table_hbm.at[idx_v.at[b]],
                                     rows0.at[pl.ds(0, 128)], g0).wait()
                    pltpu.sync_copy(rows0.at[pl.ds(0, 128)],
                                    out_hbm.at[pl.ds((b0 + b) * 128, 128)])
            nboff += nper * 32

    return k(table, idx2d)


def _sc_scatter_add(rows3, idx2d, zeros_tile):
    nbs = [r.shape[0] // 128 for r in rows3]
    assert sum(nbs) == idx2d.shape[0]
    W = KS * 128

    @functools.partial(
        pl.kernel,
        out_type=jax.ShapeDtypeStruct((N1P, U), jnp.float32),
        mesh=plsc.VectorSubcoreMesh(**_MESH),
        scratch_types=[
            pltpu.VMEM((KS, 128), jnp.int32),
            pltpu.VMEM((KS, 128), jnp.int32),
            pltpu.VMEM((W, CW), jnp.float32),
            pltpu.VMEM((W, CW), jnp.float32),
            pltpu.VMEM_SHARED((N1P, CW), jnp.float32),
            pltpu.SemaphoreType.DMA,
            pltpu.SemaphoreType.DMA,
            pltpu.SemaphoreType.DMA,
            pltpu.SemaphoreType.DMA,
        ],
        compiler_params=pltpu.CompilerParams(use_tc_tiling_on_sc=False),
    )
    def k(r2_hbm, r3_hbm, r4_hbm, idx_hbm, zeros_hbm, acc_hbm,
          idx0, idx1, rows0, rows1, acc_sh, c0, c1, a0, a1):
        c = lax.axis_index("c")
        s = lax.axis_index("s")
        idxs_v = (idx0, idx1)
        rows_v = (rows0, rows1)
        csem = (c0, c1)
        asem = (a0, a1)
        for p in range(2):
            col0 = (c * 2 + p) * CW
            pltpu.sync_copy(zeros_hbm,
                            acc_sh.at[pl.ds(s * ROWS_PER_TILE, ROWS_PER_TILE)])
            plsc.subcore_barrier()

            nboff = 0
            for rows_hbm, nb in zip((r2_hbm, r3_hbm, r4_hbm), nbs):
                nper = nb // 16
                nmain = nper // (2 * KS)
                b0 = nboff + s * nper
                r0 = s * nper

                @pl.loop(0, nmain)
                def _(j):
                    ld = []
                    for d in range(2):
                        bb = (j * 2 + d) * KS
                        ld.append((
                            pltpu.async_copy(idx_hbm.at[pl.ds(b0 + bb, KS)],
                                             idxs_v[d], csem[d]),
                            pltpu.async_copy(
                                rows_hbm.at[pl.ds((r0 + bb) * 128, W),
                                            pl.ds(col0, CW)],
                                rows_v[d], csem[d]),
                        ))
                    adds = []
                    for d in range(2):
                        for cp in ld[d]:
                            cp.wait()
                        adds.extend(
                            pltpu.async_copy(rows_v[d].at[pl.ds(kk * 128, 128)],
                                             acc_sh.at[idxs_v[d].at[kk]],
                                             asem[d], add=True)
                            for kk in range(KS))
                    for cp in adds:
                        cp.wait()

                if nmain * 2 * KS < nper:
                    @pl.loop(nmain * 2 * KS, nper)
                    def _(b):
                        pltpu.sync_copy(idx_hbm.at[pl.ds(b0 + b, 1)],
                                        idx0.at[pl.ds(0, 1)])
                        pltpu.sync_copy(
                            rows_hbm.at[pl.ds((r0 + b) * 128, 128),
                                        pl.ds(col0, CW)],
                            rows0.at[pl.ds(0, 128)])
                        pltpu.async_copy(rows0.at[pl.ds(0, 128)],
                                         acc_sh.at[idx0.at[0]], a0,
                                         add=True).wait()
                nboff += nb

            plsc.subcore_barrier()
            pltpu.sync_copy(
                acc_sh.at[pl.ds(s * ROWS_PER_TILE, ROWS_PER_TILE)],
                acc_hbm.at[pl.ds(s * ROWS_PER_TILE, ROWS_PER_TILE),
                           pl.ds(col0, CW)],
            )
            plsc.subcore_barrier()

    return k(rows3[0], rows3[1], rows3[2], idx2d, zeros_tile)


_R = 512


def _tc_fin(h0p, w, b):
    def body(x_ref, w_ref, b_ref, o_ref):
        o_ref[...] = jnp.tanh(_dot(x_ref[...], w_ref[...]) + b_ref[...])

    return pl.pallas_call(
        body,
        grid=(N1P // _R,),
        in_specs=[
            pl.BlockSpec((_R, U), lambda i: (i, 0)),
            pl.BlockSpec((U, U), lambda i: (0, 0)),
            pl.BlockSpec((1, U), lambda i: (0, 0)),
        ],
        out_specs=pl.BlockSpec((_R, U), lambda i: (i, 0)),
        out_shape=jax.ShapeDtypeStruct((N1P, U), jnp.float32),
    )(h0p, w, b)


def _tc_gru(m, wih, whh, bih, bhh, wd1g, T, ngp, want_hg):
    def body(m_ref, wih_ref, whh_ref, bih_ref, bhh_ref, wd1_ref, y_ref,
             *maybe_hg):
        wih_v = wih_ref[...]
        whh_v = whh_ref[...]
        bih_v = bih_ref[...]
        bhh_v = bhh_ref[...]
        wd1_v = wd1_ref[...]
        h = None
        for t in range(T):
            gi = _dot(m_ref[t], wih_v) + bih_v
            gh = bhh_v if h is None else _dot(h, whh_v) + bhh_v
            r = jax.nn.sigmoid(gi[:, 0:U] + gh[..., 0:U])
            z = jax.nn.sigmoid(gi[:, U:2 * U] + gh[..., U:2 * U])
            n = jnp.tanh(gi[:, 2 * U:] + r * gh[..., 2 * U:])
            h = n - z * n if h is None else (1.0 - z) * n + z * h
            y_ref[t] = _dot(h, wd1_v)
        if want_hg:
            maybe_hg[0][...] = h

    out_shape = [jax.ShapeDtypeStruct((T, ngp, U), jnp.float32)]
    out_specs = [pl.BlockSpec((T, _R, U), lambda i: (0, i, 0))]
    if want_hg:
        out_shape.append(jax.ShapeDtypeStruct((ngp, U), jnp.float32))
        out_specs.append(pl.BlockSpec((_R, U), lambda i: (i, 0)))
    return pl.pallas_call(
        body,
        grid=(ngp // _R,),
        in_specs=[
            pl.BlockSpec((T, _R, U), lambda i: (0, i, 0)),
            pl.BlockSpec((U, 3 * U), lambda i: (0, 0)),
            pl.BlockSpec((U, 3 * U), lambda i: (0, 0)),
            pl.BlockSpec((1, 3 * U), lambda i: (0, 0)),
            pl.BlockSpec((1, 3 * U), lambda i: (0, 0)),
            pl.BlockSpec((U, U), lambda i: (0, 0)),
        ],
        out_specs=out_specs,
        out_shape=out_shape,
    )(m, wih, whh, bih, bhh, wd1g)


def _tc_combine(h, acc, w_h, bd1, wd2, bd2):
    def body(h_ref, a_ref, wh_ref, b1_ref, wd2_ref, b2_ref, o_ref):
        t = _dot(h_ref[...], wh_ref[...]) + a_ref[...] + b1_ref[...]
        o_ref[...] = jnp.tanh(_dot(jnp.tanh(t), wd2_ref[...]) + b2_ref[...])

    rspec = pl.BlockSpec((_R, U), lambda i: (i, 0))
    wspec = pl.BlockSpec((U, U), lambda i: (0, 0))
    bspec = pl.BlockSpec((1, U), lambda i: (0, 0))
    return pl.pallas_call(
        body,
        grid=(N1P // _R,),
        in_specs=[rspec, rspec, wspec, bspec, wspec, bspec],
        out_specs=rspec,
        out_shape=jax.ShapeDtypeStruct((N1P, U), jnp.float32),
    )(h, acc, w_h, bd1, wd2, bd2)


def _tc_readout(x3, t_idx, w1, b1, w2p, b2p):
    np_rows = x3.shape[1]

    def body(x_ref, w1_ref, b1_ref, w2_ref, b2_ref, o_ref):
        t = _dot(x_ref[0], w1_ref[...]) + b1_ref[...]
        o_ref[...] = _dot(t, w2_ref[...]) + b2_ref[...]

    return pl.pallas_call(
        body,
        grid=(np_rows // _R,),
        in_specs=[
            pl.BlockSpec((1, _R, U), lambda i: (t_idx, i, 0)),
            pl.BlockSpec((U, U), lambda i: (0, 0)),
            pl.BlockSpec((1, U), lambda i: (0, 0)),
            pl.BlockSpec((U, 8), lambda i: (0, 0)),
            pl.BlockSpec((1, 8), lambda i: (0, 0)),
        ],
        out_specs=pl.BlockSpec((_R, 8), lambda i: (i, 0)),
        out_shape=jax.ShapeDtypeStruct((np_rows, 8), jnp.float32),
    )(x3, w1, b1, w2p, b2p)


def _layer(h, L, p, idx_all, zeros_tile, want_hg):
    wd1 = p[L + "_Wd1"]
    nbs = [T * ngp // 128 for (T, _, ngp) in GDEFS.values()]
    ms = _sc_gather(h, idx_all, nbs)
    hgs = {}
    ys = []
    for gi, (name, (T, _, ngp)) in enumerate(GDEFS.items()):
        m = ms[gi].reshape(T, ngp, U)
        outs = _tc_gru(m, p[L + "_Wih"], p[L + "_Whh"],
                       p[L + "_bih"][None, :], p[L + "_bhh"][None, :],
                       wd1[U * (gi + 1):U * (gi + 2)], T, ngp, want_hg)
        ys.append(outs[0].reshape(T * ngp, U))
        if want_hg:
            hgs[name] = outs[1]
    acc = _sc_scatter_add(ys, idx_all, zeros_tile)
    hnew = _tc_combine(h, acc, wd1[0:U],
                       p[L + "_bd1"][None, :], p[L + "_Wd2"], p[L + "_bd2"][None, :])
    return hnew, hgs


def kernel(h0, params, g2_idx, g3_idx, g4_idx):
    p = params
    idxs = {"g2": g2_idx, "g3": g3_idx, "g4": g4_idx}

    idx2ds = []
    for name, (T, ng, ngp) in GDEFS.items():
        it = jnp.full((T, ngp), DUMP, jnp.int32)
        it = it.at[:, :ng].set(idxs[name].astype(jnp.int32).T)
        idx2ds.append(it.reshape(-1, 128))
    idx_all = jnp.concatenate(idx2ds, axis=0)

    h0p = jnp.pad(h0, ((0, N1P - N1), (0, U - h0.shape[1])))
    finw = jnp.pad(p["fin_W"], ((0, U - p["fin_W"].shape[0]), (0, 0)))
    zeros_tile = jnp.zeros((ROWS_PER_TILE, CW), jnp.float32)

    h = _tc_fin(h0p, finw, p["fin_b"][None, :])
    h, _ = _layer(h, "d0", p, idx_all, zeros_tile, False)
    h, hgs = _layer(h, "d2", p, idx_all, zeros_tile, True)

    outs = []
    ro_in = {
        "atom": (h, N1),
        "bond": (hgs["g2"], N1),
        "angle": (hgs["g3"], GDEFS["g3"][1]),
        "torsion": (hgs["g4"], GDEFS["g4"][1]),
    }
    for term, (x2, nreal) in ro_in.items():
        w2p = jnp.pad(p["fr_" + term + "_W2"], ((0, 0), (0, 6)))
        b2p = jnp.pad(p["fr_" + term + "_b2"], ((0, 6)))[None, :]
        o = _tc_readout(x2[None], 0, p["fr_" + term + "_W1"],
                        p["fr_" + term + "_b1"][None, :], w2p, b2p)
        outs.append(o[:nreal, :2])
    return jnp.concatenate(outs, axis=0)
